# baseline (speedup 1.0000x reference)
import functools

import jax
import jax.numpy as jnp
from jax import lax
from jax.experimental import pallas as pl
from jax.experimental.pallas import tpu as pltpu
from jax.experimental.pallas import tpu_sc as plsc

D_MODEL = 2048
B_TOTAL = 4 * 4096
NC = 2
NS = 16
NW = NC * NS
B_PER_W = B_TOTAL // NW
CHUNK = 16
N_CHUNKS = B_PER_W // CHUNK
N_PAIRS = N_CHUNKS // 2


def _make_gather():
    mesh = plsc.VectorSubcoreMesh(core_axis_name="c", subcore_axis_name="s")

    @functools.partial(
        pl.kernel,
        mesh=mesh,
        out_type=jax.ShapeDtypeStruct((B_TOTAL, D_MODEL), jnp.float32),
        scratch_types=[
            pltpu.VMEM((B_PER_W,), jnp.int32),
            pltpu.VMEM((CHUNK, D_MODEL), jnp.float32),
            pltpu.VMEM((CHUNK, D_MODEL), jnp.float32),
            pltpu.SemaphoreType.DMA,
            pltpu.SemaphoreType.DMA,
        ],
    )
    def k(idx_hbm, table_hbm, out_hbm, idx_v, buf0, buf1, gsem, osem):
        wid = lax.axis_index("s") * NC + lax.axis_index("c")
        base = wid * B_PER_W
        pltpu.sync_copy(idx_hbm.at[pl.ds(base, B_PER_W)], idx_v)
        bufs = (buf0, buf1)

        def fire_g(c, buf):
            start = pl.multiple_of(c * CHUNK, 8)
            pltpu.async_copy(table_hbm.at[idx_v.at[pl.ds(start, CHUNK)]], buf, gsem)

        def wait_g(buf):
            pltpu.make_async_copy(table_hbm.at[pl.ds(0, CHUNK)], buf, gsem).wait()

        def fire_o(c, buf):
            pltpu.async_copy(buf, out_hbm.at[pl.ds(base + c * CHUNK, CHUNK)], osem)

        def wait_o(buf):
            pltpu.make_async_copy(buf, out_hbm.at[pl.ds(base, CHUNK)], osem).wait()

        fire_g(0, buf0)
        wait_g(buf0)
        fire_o(0, buf0)
        fire_g(1, buf1)

        def step(c, cur, nxt):
            wait_g(cur)
            fire_o(c, cur)
            wait_o(nxt)
            fire_g(c + 1, nxt)

        def pair_body(i, carry):
            c0 = 2 * i + 1
            step(c0, buf1, buf0)
            step(c0 + 1, buf0, buf1)
            return carry

        lax.fori_loop(0, (N_CHUNKS - 2) // 2, pair_body, 0)

        wait_g(buf1)
        fire_o(N_CHUNKS - 1, buf1)
        wait_o(buf0)
        wait_o(buf1)

    return k


_gather = _make_gather()


def kernel(tokens, W_E):
    idx = tokens.reshape(-1)
    out = _gather(idx, W_E)
    return out.reshape(tokens.shape[0], tokens.shape[1], W_E.shape[0])
